# baseline (speedup 1.0000x reference)
import functools

import jax
import jax.numpy as jnp
from jax import lax
from jax.experimental import pallas as pl
from jax.experimental.pallas import tpu as pltpu
from jax.experimental.pallas import tpu_sc as plsc

_K = 20
_KP = 24


def _feat_body(x_ref, w_ref, b_ref, o_ref):
    x = x_ref[0]
    o_ref[0] = (
        jnp.dot(x, w_ref[...], preferred_element_type=jnp.float32)
        + b_ref[...]
    )


def _features(X, W, b):
    E, N, Din = X.shape
    EMB = W.shape[1]
    return pl.pallas_call(
        _feat_body,
        grid=(E,),
        in_specs=[
            pl.BlockSpec((1, N, Din), lambda e: (e, 0, 0)),
            pl.BlockSpec((Din, EMB), lambda e: (0, 0)),
            pl.BlockSpec((1, EMB), lambda e: (0, 0)),
        ],
        out_specs=pl.BlockSpec((1, N, EMB), lambda e: (e, 0, 0)),
        out_shape=jax.ShapeDtypeStruct((E, N, EMB), jnp.float32),
    )(X, W, b.reshape(1, EMB))


def _topk_body(fr_ref, fa_ref, o_ref, *, k, chunk=512):
    fr = fr_ref[0]
    fa = fa_ref[0]
    R = fr.shape[0]
    N = fa.shape[0]
    C = min(chunk, N)
    NCH = N // C
    sqr = jnp.sum(fr * fr, axis=1)
    sqa = jnp.sum(fa * fa, axis=1)
    prod = lax.dot_general(
        fr, fa, (((1,), (1,)), ((), ())), preferred_element_type=jnp.float32
    )
    dist = sqr[:, None] + sqa[None, :] - 2.0 * prod
    rows = pl.program_id(1) * R + lax.broadcasted_iota(jnp.int32, (R, N), 0)
    cols = lax.broadcasted_iota(jnp.int32, (R, N), 1)
    dist = jnp.where(rows == cols, dist + 1e10, dist)
    off = pl.program_id(0) * N
    curs = [dist[:, c * C:(c + 1) * C] for c in range(NCH)]
    cio = [cols[:, c * C:(c + 1) * C] for c in range(NCH)]
    mc = [jnp.min(cu, axis=1) for cu in curs]
    inf = jnp.float32(jnp.inf)

    def round_body(t, carry):
        curs, mc = carry
        m = functools.reduce(jnp.minimum, mc)
        cands = [
            jnp.min(jnp.where(curs[c] == m[:, None], cio[c], N), axis=1)
            for c in range(NCH)
        ]
        it = functools.reduce(jnp.minimum, cands)
        o_ref[0, pl.ds(t, 1), :] = (it + off)[None, :]
        ncurs, nmc = [], []
        for c in range(NCH):
            masked = jnp.where(cio[c] == it[:, None], inf, curs[c])
            ncurs.append(masked)
            nmc.append(jnp.min(masked, axis=1))
        return ncurs, nmc

    lax.fori_loop(0, k, round_body, (curs, mc))


def _topk(F, k, kp, row_block=256):
    E, N, EMB = F.shape
    R = min(row_block, N)
    return pl.pallas_call(
        functools.partial(_topk_body, k=k),
        grid=(E, N // R),
        in_specs=[
            pl.BlockSpec((1, R, EMB), lambda e, j: (e, j, 0)),
            pl.BlockSpec((1, N, EMB), lambda e, j: (e, 0, 0)),
        ],
        out_specs=pl.BlockSpec((1, kp, R), lambda e, j: (e, 0, j)),
        out_shape=jax.ShapeDtypeStruct((E, kp, N), jnp.int32),
    )(F, F)


def _make_agg(M, Dx, K, KP, N):
    info = plsc.get_sparse_core_info()
    NC = info.num_cores
    NW = NC * info.num_subcores
    npw = M // NW
    G = 128
    NG = npw // G
    NBUF = 4
    rounds = K // NBUF
    mesh = plsc.VectorSubcoreMesh(core_axis_name="c", subcore_axis_name="s")

    @functools.partial(
        pl.kernel,
        mesh=mesh,
        out_type=jax.ShapeDtypeStruct((M, Dx), jnp.float32),
        scratch_types=[
            pltpu.VMEM((KP, G), jnp.int32),
            pltpu.VMEM((G, Dx), jnp.float32),
            pltpu.VMEM((NBUF, G, Dx), jnp.float32),
            pltpu.SemaphoreType.DMA,
        ],
    )
    def agg(table_hbm, idx_hbm, out_hbm, idx_v, acc_v, bufs_v, sem):
        c = lax.axis_index("c")
        s = lax.axis_index("s")
        wid = s * NC + c
        node0 = wid * npw
        e = node0 // N
        for g in range(NG):
            nbase = node0 + g * G
            col = nbase - e * N
            pltpu.sync_copy(
                idx_hbm.at[pl.ds(e * KP, KP), pl.ds(col, G)], idx_v
            )
            pltpu.sync_copy(table_hbm.at[pl.ds(nbase, G)], acc_v)
            for r in range(rounds):
                cps = [
                    pltpu.async_copy(
                        table_hbm.at[idx_v.at[r * NBUF + j]],
                        bufs_v.at[j],
                        sem,
                    )
                    for j in range(NBUF)
                ]
                for cp in cps:
                    cp.wait()

                def row_body(rr, _):
                    for ch in range(Dx // 16):
                        sl = pl.ds(ch * 16, 16)
                        v = acc_v[rr, sl]
                        for j in range(NBUF):
                            v = v + bufs_v[j, rr, sl]
                        acc_v[rr, sl] = v
                    return 0

                lax.fori_loop(0, G, row_body, 0)
            pltpu.sync_copy(acc_v, out_hbm.at[pl.ds(nbase, G)])

    return agg


def _mlp_body(h_ref, wa_ref, ba_ref, wb_ref, bb_ref, o_ref):
    h = h_ref[...]
    z = jnp.maximum(
        jnp.dot(h, wa_ref[...], preferred_element_type=jnp.float32)
        + ba_ref[...],
        0.0,
    )
    o_ref[...] = (
        jnp.dot(z, wb_ref[...], preferred_element_type=jnp.float32)
        + bb_ref[...]
    )


def _mlp(h, Wa, ba, Wb, bb, row_block=2048):
    M, Din = h.shape
    H1 = Wa.shape[1]
    H2 = Wb.shape[1]
    R = min(row_block, M)
    return pl.pallas_call(
        _mlp_body,
        grid=(M // R,),
        in_specs=[
            pl.BlockSpec((R, Din), lambda i: (i, 0)),
            pl.BlockSpec((Din, H1), lambda i: (0, 0)),
            pl.BlockSpec((1, H1), lambda i: (0, 0)),
            pl.BlockSpec((H1, H2), lambda i: (0, 0)),
            pl.BlockSpec((1, H2), lambda i: (0, 0)),
        ],
        out_specs=pl.BlockSpec((R, H2), lambda i: (i, 0)),
        out_shape=jax.ShapeDtypeStruct((M, H2), jnp.float32),
    )(h, Wa, ba.reshape(1, H1), Wb, bb.reshape(1, H2))


def _sim_body(z1_ref, z2_ref, o_ref):
    z1 = z1_ref[0]
    z2 = z2_ref[0]
    s = lax.dot_general(
        z1, z2, (((1,), (1,)), ((), ())), preferred_element_type=jnp.float32
    )
    m = jnp.max(s, axis=1, keepdims=True)
    e = jnp.exp(s - m)
    o_ref[0] = e / jnp.sum(e, axis=1, keepdims=True)


def _sim_softmax(Z, B, row_block=256):
    E, N, HID = Z.shape
    R = row_block
    return pl.pallas_call(
        _sim_body,
        grid=(B, N // R),
        in_specs=[
            pl.BlockSpec((1, R, HID), lambda b, j: (b, j, 0)),
            pl.BlockSpec((1, N, HID), lambda b, j: (B + b, 0, 0)),
        ],
        out_specs=pl.BlockSpec((1, R, N), lambda b, j: (b, j, 0)),
        out_shape=jax.ShapeDtypeStruct((B, N, N), jnp.float32),
    )(Z, Z)


def kernel(pc1, pc2, W_feat, b_feat, W1a, b1a, W1b, b1b, W2a, b2a, W2b, b2b):
    B, Din, N = pc1.shape
    EMB = W_feat.shape[1]
    HID = W1a.shape[1]
    E = 2 * B

    X = jnp.concatenate([pc1, pc2], axis=0).transpose(0, 2, 1)
    F = _features(X, W_feat, b_feat)
    idxT = _topk(F, _K, _KP)

    Ff = F.reshape(E * N, EMB)
    idxf = idxT.reshape(E * _KP, N)
    h1 = _make_agg(E * N, EMB, _K, _KP, N)(Ff, idxf)
    pad = EMB - HID
    W1b_p = jnp.pad(W1b, ((0, 0), (0, pad)))
    b1b_p = jnp.pad(b1b, (0, pad))
    W2a_p = jnp.pad(W2a, ((0, pad), (0, 0)))
    z = _mlp(h1, W1a, b1a, W1b_p, b1b_p)
    h2 = _make_agg(E * N, EMB, _K, _KP, N)(z, idxf)
    z = _mlp(h2, W2a_p, b2a, W2b, b2b)

    Z = z.reshape(E, N, HID)
    return _sim_softmax(Z, B)


# speedup vs baseline: 1.8899x; 1.8899x over previous
import functools

import jax
import jax.numpy as jnp
from jax import lax
from jax.experimental import pallas as pl
from jax.experimental.pallas import tpu as pltpu
from jax.experimental.pallas import tpu_sc as plsc

_K = 20
_KP = 24


def _feat_body(x_ref, w_ref, b_ref, o_ref):
    x = x_ref[0]
    o_ref[0] = (
        jnp.dot(x, w_ref[...], preferred_element_type=jnp.float32)
        + b_ref[...]
    )


def _features(X, W, b):
    E, N, Din = X.shape
    EMB = W.shape[1]
    return pl.pallas_call(
        _feat_body,
        grid=(E,),
        in_specs=[
            pl.BlockSpec((1, N, Din), lambda e: (e, 0, 0)),
            pl.BlockSpec((Din, EMB), lambda e: (0, 0)),
            pl.BlockSpec((1, EMB), lambda e: (0, 0)),
        ],
        out_specs=pl.BlockSpec((1, N, EMB), lambda e: (e, 0, 0)),
        out_shape=jax.ShapeDtypeStruct((E, N, EMB), jnp.float32),
    )(X, W, b.reshape(1, EMB))


def _topk_body(fr_ref, fa_ref, o_ref, *, k, chunk=512):
    fr = fr_ref[0]
    fa = fa_ref[0]
    R = fr.shape[0]
    N = fa.shape[0]
    C = min(chunk, N)
    NCH = N // C
    sqr = jnp.sum(fr * fr, axis=1)
    sqa = jnp.sum(fa * fa, axis=1)
    prod = lax.dot_general(
        fr, fa, (((1,), (1,)), ((), ())), preferred_element_type=jnp.float32
    )
    dist = sqr[:, None] + sqa[None, :] - 2.0 * prod
    rows = pl.program_id(1) * R + lax.broadcasted_iota(jnp.int32, (R, N), 0)
    cols = lax.broadcasted_iota(jnp.int32, (R, N), 1)
    dist = jnp.where(rows == cols, dist + 1e10, dist)
    off = pl.program_id(0) * N
    cur = dist
    for t in range(k):
        m = jnp.min(cur, axis=1)
        eqm = cur == m[:, None]
        cand = jnp.where(eqm, cols, N)
        it = jnp.min(cand, axis=1)
        o_ref[0, t, :] = it + off
        if t + 1 < k:
            cur = jnp.where(cols == it[:, None], jnp.float32(jnp.inf), cur)


def _topk(F, k, kp, row_block=256):
    E, N, EMB = F.shape
    R = min(row_block, N)
    return pl.pallas_call(
        functools.partial(_topk_body, k=k),
        grid=(E, N // R),
        in_specs=[
            pl.BlockSpec((1, R, EMB), lambda e, j: (e, j, 0)),
            pl.BlockSpec((1, N, EMB), lambda e, j: (e, 0, 0)),
        ],
        out_specs=pl.BlockSpec((1, kp, R), lambda e, j: (e, 0, j)),
        out_shape=jax.ShapeDtypeStruct((E, kp, N), jnp.int32),
    )(F, F)


def _make_agg(M, Dx, K, KP, N):
    info = plsc.get_sparse_core_info()
    NC = info.num_cores
    NW = NC * info.num_subcores
    npw = M // NW
    G = 128
    NG = npw // G
    NBUF = 2
    BANKS = 2
    rounds = K // NBUF
    mesh = plsc.VectorSubcoreMesh(core_axis_name="c", subcore_axis_name="s")

    @functools.partial(
        pl.kernel,
        mesh=mesh,
        out_type=jax.ShapeDtypeStruct((M, Dx), jnp.float32),
        scratch_types=[
            pltpu.VMEM((KP, G), jnp.int32),
            pltpu.VMEM((G, Dx), jnp.float32),
            pltpu.VMEM((BANKS * NBUF, G, Dx), jnp.float32),
            pltpu.SemaphoreType.DMA,
            pltpu.SemaphoreType.DMA,
        ],
    )
    def agg(table_hbm, idx_hbm, out_hbm, idx_v, acc_v, bufs_v, sem0, sem1):
        c = lax.axis_index("c")
        s = lax.axis_index("s")
        wid = s * NC + c
        node0 = wid * npw
        e = node0 // N
        sems = [sem0, sem1]
        for g in range(NG):
            nbase = node0 + g * G
            col = nbase - e * N
            pltpu.sync_copy(
                idx_hbm.at[pl.ds(e * KP, KP), pl.ds(col, G)], idx_v
            )
            pltpu.sync_copy(table_hbm.at[pl.ds(nbase, G)], acc_v)

            def fire(r, bank):
                return [
                    pltpu.async_copy(
                        table_hbm.at[idx_v.at[r * NBUF + j]],
                        bufs_v.at[bank * NBUF + j],
                        sems[bank],
                    )
                    for j in range(NBUF)
                ]

            pending = fire(0, 0)
            for r in range(rounds):
                bank = r % BANKS
                for cp in pending:
                    cp.wait()
                if r + 1 < rounds:
                    nxt = fire(r + 1, (r + 1) % BANKS)

                def row_body(rr, _, bank=bank):
                    for u in range(2):
                        row = rr * 2 + u
                        for ch in range(Dx // 16):
                            sl = pl.ds(ch * 16, 16)
                            v = acc_v[row, sl]
                            for j in range(NBUF):
                                v = v + bufs_v[bank * NBUF + j, row, sl]
                            acc_v[row, sl] = v
                    return 0

                lax.fori_loop(0, G // 2, row_body, 0)
                pending = nxt if r + 1 < rounds else []
            pltpu.sync_copy(acc_v, out_hbm.at[pl.ds(nbase, G)])

    return agg


def _mlp_body(h_ref, wa_ref, ba_ref, wb_ref, bb_ref, o_ref):
    h = h_ref[...]
    z = jnp.maximum(
        jnp.dot(h, wa_ref[...], preferred_element_type=jnp.float32)
        + ba_ref[...],
        0.0,
    )
    o_ref[...] = (
        jnp.dot(z, wb_ref[...], preferred_element_type=jnp.float32)
        + bb_ref[...]
    )


def _mlp(h, Wa, ba, Wb, bb, row_block=2048):
    M, Din = h.shape
    H1 = Wa.shape[1]
    H2 = Wb.shape[1]
    R = min(row_block, M)
    return pl.pallas_call(
        _mlp_body,
        grid=(M // R,),
        in_specs=[
            pl.BlockSpec((R, Din), lambda i: (i, 0)),
            pl.BlockSpec((Din, H1), lambda i: (0, 0)),
            pl.BlockSpec((1, H1), lambda i: (0, 0)),
            pl.BlockSpec((H1, H2), lambda i: (0, 0)),
            pl.BlockSpec((1, H2), lambda i: (0, 0)),
        ],
        out_specs=pl.BlockSpec((R, H2), lambda i: (i, 0)),
        out_shape=jax.ShapeDtypeStruct((M, H2), jnp.float32),
    )(h, Wa, ba.reshape(1, H1), Wb, bb.reshape(1, H2))


def _sim_body(z1_ref, z2_ref, o_ref):
    z1 = z1_ref[0]
    z2 = z2_ref[0]
    s = lax.dot_general(
        z1, z2, (((1,), (1,)), ((), ())), preferred_element_type=jnp.float32
    )
    m = jnp.max(s, axis=1, keepdims=True)
    e = jnp.exp(s - m)
    o_ref[0] = e / jnp.sum(e, axis=1, keepdims=True)


def _sim_softmax(Z, B, row_block=256):
    E, N, HID = Z.shape
    R = row_block
    return pl.pallas_call(
        _sim_body,
        grid=(B, N // R),
        in_specs=[
            pl.BlockSpec((1, R, HID), lambda b, j: (b, j, 0)),
            pl.BlockSpec((1, N, HID), lambda b, j: (B + b, 0, 0)),
        ],
        out_specs=pl.BlockSpec((1, R, N), lambda b, j: (b, j, 0)),
        out_shape=jax.ShapeDtypeStruct((B, N, N), jnp.float32),
    )(Z, Z)


def kernel(pc1, pc2, W_feat, b_feat, W1a, b1a, W1b, b1b, W2a, b2a, W2b, b2b):
    B, Din, N = pc1.shape
    EMB = W_feat.shape[1]
    HID = W1a.shape[1]
    E = 2 * B

    X = jnp.concatenate([pc1, pc2], axis=0).transpose(0, 2, 1)
    F = _features(X, W_feat, b_feat)
    idxT = _topk(F, _K, _KP)

    Ff = F.reshape(E * N, EMB)
    idxf = idxT.reshape(E * _KP, N)
    h1 = _make_agg(E * N, EMB, _K, _KP, N)(Ff, idxf)
    pad = EMB - HID
    W1b_p = jnp.pad(W1b, ((0, 0), (0, pad)))
    b1b_p = jnp.pad(b1b, (0, pad))
    W2a_p = jnp.pad(W2a, ((0, pad), (0, 0)))
    z = _mlp(h1, W1a, b1a, W1b_p, b1b_p)
    h2 = _make_agg(E * N, EMB, _K, _KP, N)(z, idxf)
    z = _mlp(h2, W2a_p, b2a, W2b, b2b)

    Z = z.reshape(E, N, HID)
    return _sim_softmax(Z, B)
